# baseline (speedup 1.0000x reference)
import functools

import jax
import jax.numpy as jnp
import numpy as np
from jax import lax
from jax.experimental import pallas as pl
from jax.experimental.pallas import tpu as pltpu
from jax.experimental.pallas import tpu_sc as plsc

N, E, D = 10000, 160000, 256
NP = 10240
EP = 163840
NW = 32
EW = EP // NW
CH = 64
NCHUNK = EW // CH
NBUF = 3
NFULL = (NCHUNK // NBUF) * NBUF
CW = 128
NPASS = 2 * D // CW
RPT = NP // 16
BLK = 256
GRID = NP // BLK

_mesh = plsc.VectorSubcoreMesh(core_axis_name="c", subcore_axis_name="s")

with jax.default_device(jax.devices("cpu")[0]):
    _SAMPLE = np.asarray(
        jax.random.normal(jax.random.key(42), (N, D), jnp.float32))


@functools.partial(
    pl.kernel,
    out_type=jax.ShapeDtypeStruct((2, NP), jnp.float32),
    mesh=_mesh,
    scratch_types=[
        pltpu.VMEM((EW,), jnp.int32),
        pltpu.VMEM((NP,), jnp.float32),
        pltpu.VMEM((16, RPT), jnp.float32),
        pltpu.VMEM_SHARED((16, NP), jnp.float32),
    ],
    compiler_params=pltpu.CompilerParams(needs_layout_passes=False),
)
def _deg_call(row_hbm, deg_out, idx_v, hist_v, stage_v, shared):
    c = lax.axis_index("c")
    s = lax.axis_index("s")
    w = c * 16 + s
    zero = jnp.zeros((16,), jnp.float32)

    @pl.loop(0, NP // 16)
    def _zero(i):
        hist_v[pl.ds(i * 16, 16)] = zero

    pltpu.sync_copy(row_hbm.at[pl.ds(w * EW, EW)], idx_v)
    ones = jnp.ones((16,), jnp.float32)

    @pl.loop(0, EW // 16)
    def _hist(g):
        idx = idx_v[pl.ds(g * 16, 16)]
        plsc.addupdate_scatter(hist_v, [idx], ones)

    pltpu.sync_copy(hist_v, shared.at[s])
    plsc.subcore_barrier()
    for t in range(16):
        pltpu.sync_copy(shared.at[t, pl.ds(s * RPT, RPT)], stage_v.at[t])

    @pl.loop(0, RPT // 16)
    def _reduce(g):
        acc = stage_v[0, pl.ds(g * 16, 16)]
        for t in range(1, 16):
            acc = acc + stage_v[t, pl.ds(g * 16, 16)]
        hist_v[pl.ds(g * 16, 16)] = acc

    pltpu.sync_copy(hist_v.at[pl.ds(0, RPT)], deg_out.at[c, pl.ds(s * RPT, RPT)])


def _elu(x):
    return jnp.where(x > 0, x, jnp.exp(jnp.minimum(x, 0.0)) - 1.0)


def _dense_body(x_ref, w0m_ref, b0m_ref, w0v_ref, b0v_ref, w1m_ref, b1m_ref,
                w1v_ref, b1v_ref, d0_ref, d1_ref, *out_refs):
    bf16, f32 = jnp.bfloat16, jnp.float32
    x = x_ref[...].astype(bf16)
    m0 = _elu(jnp.dot(x, w0m_ref[...].astype(bf16),
                      preferred_element_type=f32) + b0m_ref[...])
    v0 = jax.nn.relu(jnp.dot(x, w0v_ref[...].astype(bf16),
                             preferred_element_type=f32) + b0v_ref[...])
    m1 = _elu(jnp.dot(m0.astype(bf16), w1m_ref[...].astype(bf16),
                      preferred_element_type=f32) + b1m_ref[...])
    v1 = jax.nn.relu(jnp.dot(v0.astype(bf16), w1v_ref[...].astype(bf16),
                             preferred_element_type=f32) + b1v_ref[...]) + 1e-6
    att = jnp.exp(-v1)
    deg = d0_ref[...] + d1_ref[...] + 1.0
    dinv1 = 1.0 / deg
    dh = lax.rsqrt(deg)
    mean_s = m1 * att * dh
    var_s = v1 * (att * att) * dinv1
    for q in range(NPASS // 2):
        out_refs[q][...] = mean_s[:, q * CW:(q + 1) * CW]
        out_refs[NPASS // 2 + q][...] = var_s[:, q * CW:(q + 1) * CW]


def _dense_call(X, W0m, b0m, W0v, b0v, W1m, b1m, W1v, b1v, d0, d1):
    quarter = jax.ShapeDtypeStruct((NP, CW), jnp.float32)
    wspec = pl.BlockSpec((D, D), lambda i: (0, 0))
    bspec = pl.BlockSpec((1, D), lambda i: (0, 0))
    dspec = pl.BlockSpec((BLK, 1), lambda i: (i, 0))
    qspec = pl.BlockSpec((BLK, CW), lambda i: (i, 0))
    return pl.pallas_call(
        _dense_body,
        grid=(GRID,),
        in_specs=[pl.BlockSpec((BLK, D), lambda i: (i, 0)),
                  wspec, bspec, wspec, bspec, wspec, bspec, wspec, bspec,
                  dspec, dspec],
        out_specs=[qspec] * NPASS,
        out_shape=[quarter] * NPASS,
    )(X, W0m, b0m, W0v, b0v, W1m, b1m, W1v, b1v, d0, d1)


def _make_spmm(npass):
    @functools.partial(
        pl.kernel,
        out_type=jax.ShapeDtypeStruct((2, npass, NP, CW), jnp.float32),
        mesh=_mesh,
        scratch_types=[
            pltpu.VMEM((NCHUNK, CH), jnp.int32),
            pltpu.VMEM((NCHUNK, CH), jnp.int32),
            [pltpu.VMEM((CH, CW), jnp.float32)] * NBUF,
            [pltpu.SemaphoreType.DMA] * NBUF,
            pltpu.VMEM_SHARED((NP, CW), jnp.float32),
        ],
        compiler_params=pltpu.CompilerParams(needs_layout_passes=False),
    )
    def _spmm(row_hbm, col_hbm, *args):
        tabs = args[:npass]
        out_hbm = args[npass]
        idxc, idxr, bufs, sems, acc = args[npass + 1:]
        c = lax.axis_index("c")
        s = lax.axis_index("s")
        w = c * 16 + s
        zero = jnp.zeros((16,), jnp.float32)

        pltpu.sync_copy(col_hbm.at[w], idxc)
        pltpu.sync_copy(row_hbm.at[w], idxr)

        for p, tab in enumerate(tabs):
            @pl.loop(0, CH)
            def _zrow(r):
                for l in range(CW // 16):
                    bufs[0][r, pl.ds(l * 16, 16)] = zero

            for k in range(RPT // CH):
                pltpu.sync_copy(bufs[0], acc.at[pl.ds(s * RPT + k * CH, CH)])
            plsc.subcore_barrier()

            for b in range(NBUF):
                pltpu.async_copy(tab.at[idxc.at[b]], bufs[b], sems[b])

            @pl.loop(0, NFULL // NBUF)
            def _edges(g):
                for b in range(NBUF):
                    j = g * NBUF + b
                    pltpu.make_async_copy(tab.at[idxc.at[j]], bufs[b],
                                          sems[b]).wait()
                    pltpu.sync_copy(bufs[b], acc.at[idxr.at[j]], add=True)

                    @pl.when(j + NBUF < NCHUNK)
                    def _issue():
                        pltpu.async_copy(tab.at[idxc.at[j + NBUF]],
                                         bufs[b], sems[b])

            for j in range(NFULL, NCHUNK):
                b = j % NBUF
                pltpu.make_async_copy(tab.at[idxc.at[j]], bufs[b],
                                      sems[b]).wait()
                pltpu.sync_copy(bufs[b], acc.at[idxr.at[j]], add=True)

            plsc.subcore_barrier()
            pltpu.sync_copy(acc.at[pl.ds(s * RPT, RPT)],
                            out_hbm.at[c, p, pl.ds(s * RPT, RPT)])
            plsc.subcore_barrier()

    return _spmm


_spmm_half = _make_spmm(2)


def _cmb_body(acc_ref, f0_ref, f1_ref, d0_ref, d1_ref, o_ref):
    a = acc_ref[...]
    deg = d0_ref[...] + d1_ref[...] + 1.0
    dh = lax.rsqrt(deg)
    mean = jnp.concatenate(
        [a[0, 0] + a[1, 0] + f0_ref[...], a[0, 1] + a[1, 1] + f1_ref[...]],
        axis=1)
    o_ref[...] = mean * dh


def _cmb_call(accm, f0, f1, d0, d1):
    qspec = pl.BlockSpec((BLK, CW), lambda i: (i, 0))
    dspec = pl.BlockSpec((BLK, 1), lambda i: (i, 0))
    return pl.pallas_call(
        _cmb_body,
        grid=(GRID,),
        in_specs=[pl.BlockSpec((2, 2, BLK, CW), lambda i: (0, 0, i, 0)),
                  qspec, qspec, dspec, dspec],
        out_specs=pl.BlockSpec((BLK, D), lambda i: (i, 0)),
        out_shape=jax.ShapeDtypeStruct((NP, D), jnp.float32),
    )(accm, f0, f1, d0, d1)


def _final_body(mean_ref, acc_ref, f0_ref, f1_ref, d0_ref, d1_ref, smp_ref,
                o_ref):
    a = acc_ref[...]
    var = jnp.concatenate(
        [a[0, 0] + a[1, 0] + f0_ref[...], a[0, 1] + a[1, 1] + f1_ref[...]],
        axis=1)
    deg = d0_ref[...] + d1_ref[...] + 1.0
    dinv1 = 1.0 / deg
    out = mean_ref[...] + smp_ref[...] * jnp.sqrt(var * dinv1)
    m = jnp.max(out, axis=1, keepdims=True)
    sh = out - m
    lse = jnp.log(jnp.sum(jnp.exp(sh), axis=1, keepdims=True))
    o_ref[...] = sh - lse


def _final_call(meanc, accv, f0, f1, d0, d1, sample):
    qspec = pl.BlockSpec((BLK, CW), lambda i: (i, 0))
    dspec = pl.BlockSpec((BLK, 1), lambda i: (i, 0))
    fspec = pl.BlockSpec((BLK, D), lambda i: (i, 0))
    return pl.pallas_call(
        _final_body,
        grid=(GRID,),
        in_specs=[fspec,
                  pl.BlockSpec((2, 2, BLK, CW), lambda i: (0, 0, i, 0)),
                  qspec, qspec, dspec, dspec, fspec],
        out_specs=pl.BlockSpec((BLK, D), lambda i: (i, 0)),
        out_shape=jax.ShapeDtypeStruct((N, D), jnp.float32),
    )(meanc, accv, f0, f1, d0, d1, sample)


def kernel(X, A, W, W0m, b0m, W0v, b0v, W1m, b1m, W1v, b1v):
    del W
    pad = N + jnp.arange(EP - E, dtype=jnp.int32) % (NP - N)
    rowp = jnp.concatenate([A[0], pad])
    colp = jnp.concatenate([A[1], pad])

    deg2 = _deg_call(rowp)
    d0 = deg2[0].reshape(NP, 1)
    d1 = deg2[1].reshape(NP, 1)

    feats = _dense_call(
        X, W0m, b0m.reshape(1, D), W0v, b0v.reshape(1, D),
        W1m, b1m.reshape(1, D), W1v, b1v.reshape(1, D), d0, d1)
    mlo, mhi, vlo, vhi = feats

    row3 = rowp.reshape(NW, NCHUNK, CH)
    col3 = colp.reshape(NW, NCHUNK, CH)
    accm = _spmm_half(row3, col3, mlo, mhi)
    accv = _spmm_half(row3, col3, vlo, vhi)

    meanc = _cmb_call(accm, mlo, mhi, d0, d1)

    sample = jnp.asarray(_SAMPLE)
    return _final_call(meanc, accv, vlo, vhi, d0, d1, sample)


# speedup vs baseline: 1.0206x; 1.0206x over previous
import functools

import jax
import jax.numpy as jnp
import numpy as np
from jax import lax
from jax.experimental import pallas as pl
from jax.experimental.pallas import tpu as pltpu
from jax.experimental.pallas import tpu_sc as plsc

N, E, D = 10000, 160000, 256
NP = 10240
EP = 163840
NW = 32
EW = EP // NW
CH = 64
NCHUNK = EW // CH
NBUF = 3
NFULL = (NCHUNK // NBUF) * NBUF
CW = 128
NPASS = 2 * D // CW
RPT = NP // 16
BLK = 256
GRID = NP // BLK

_mesh = plsc.VectorSubcoreMesh(core_axis_name="c", subcore_axis_name="s")

with jax.default_device(jax.devices("cpu")[0]):
    _SAMPLE = np.asarray(
        jax.random.normal(jax.random.key(42), (N, D), jnp.float32))


@functools.partial(
    pl.kernel,
    out_type=jax.ShapeDtypeStruct((2, NP), jnp.float32),
    mesh=_mesh,
    scratch_types=[
        pltpu.VMEM((EW,), jnp.int32),
        pltpu.VMEM((NP,), jnp.float32),
        pltpu.VMEM((16, RPT), jnp.float32),
        pltpu.VMEM_SHARED((16, NP), jnp.float32),
    ],
    compiler_params=pltpu.CompilerParams(needs_layout_passes=False),
)
def _deg_call(row_hbm, deg_out, idx_v, hist_v, stage_v, shared):
    c = lax.axis_index("c")
    s = lax.axis_index("s")
    w = c * 16 + s
    zero = jnp.zeros((16,), jnp.float32)

    @pl.loop(0, NP // 16)
    def _zero(i):
        hist_v[pl.ds(i * 16, 16)] = zero

    pltpu.sync_copy(row_hbm.at[pl.ds(w * EW, EW)], idx_v)
    ones = jnp.ones((16,), jnp.float32)

    @pl.loop(0, EW // 16)
    def _hist(g):
        idx = idx_v[pl.ds(g * 16, 16)]
        plsc.addupdate_scatter(hist_v, [idx], ones)

    pltpu.sync_copy(hist_v, shared.at[s])
    plsc.subcore_barrier()
    for t in range(16):
        pltpu.sync_copy(shared.at[t, pl.ds(s * RPT, RPT)], stage_v.at[t])

    @pl.loop(0, RPT // 16)
    def _reduce(g):
        acc = stage_v[0, pl.ds(g * 16, 16)]
        for t in range(1, 16):
            acc = acc + stage_v[t, pl.ds(g * 16, 16)]
        hist_v[pl.ds(g * 16, 16)] = acc

    pltpu.sync_copy(hist_v.at[pl.ds(0, RPT)], deg_out.at[c, pl.ds(s * RPT, RPT)])


def _elu(x):
    return jnp.where(x > 0, x, jnp.exp(jnp.minimum(x, 0.0)) - 1.0)


def _dense_body(x_ref, w0m_ref, b0m_ref, w0v_ref, b0v_ref, w1m_ref, b1m_ref,
                w1v_ref, b1v_ref, d0_ref, d1_ref, *out_refs):
    bf16, f32 = jnp.bfloat16, jnp.float32
    x = x_ref[...].astype(bf16)
    m0 = _elu(jnp.dot(x, w0m_ref[...].astype(bf16),
                      preferred_element_type=f32) + b0m_ref[...])
    v0 = jax.nn.relu(jnp.dot(x, w0v_ref[...].astype(bf16),
                             preferred_element_type=f32) + b0v_ref[...])
    m1 = _elu(jnp.dot(m0.astype(bf16), w1m_ref[...].astype(bf16),
                      preferred_element_type=f32) + b1m_ref[...])
    v1 = jax.nn.relu(jnp.dot(v0.astype(bf16), w1v_ref[...].astype(bf16),
                             preferred_element_type=f32) + b1v_ref[...]) + 1e-6
    att = jnp.exp(-v1)
    deg = d0_ref[...] + d1_ref[...] + 1.0
    dinv1 = 1.0 / deg
    dh = lax.rsqrt(deg)
    mean_s = m1 * att * dh
    var_s = v1 * (att * att) * dinv1
    for q in range(NPASS // 2):
        out_refs[q][...] = mean_s[:, q * CW:(q + 1) * CW]
        out_refs[NPASS // 2 + q][...] = var_s[:, q * CW:(q + 1) * CW]


def _dense_call(X, W0m, b0m, W0v, b0v, W1m, b1m, W1v, b1v, d0, d1):
    quarter = jax.ShapeDtypeStruct((NP, CW), jnp.float32)
    wspec = pl.BlockSpec((D, D), lambda i: (0, 0))
    bspec = pl.BlockSpec((1, D), lambda i: (0, 0))
    dspec = pl.BlockSpec((BLK, 1), lambda i: (i, 0))
    qspec = pl.BlockSpec((BLK, CW), lambda i: (i, 0))
    return pl.pallas_call(
        _dense_body,
        grid=(GRID,),
        in_specs=[pl.BlockSpec((BLK, D), lambda i: (i, 0)),
                  wspec, bspec, wspec, bspec, wspec, bspec, wspec, bspec,
                  dspec, dspec],
        out_specs=[qspec] * NPASS,
        out_shape=[quarter] * NPASS,
    )(X, W0m, b0m, W0v, b0v, W1m, b1m, W1v, b1v, d0, d1)


@functools.partial(
    pl.kernel,
    out_type=jax.ShapeDtypeStruct((2, NPASS, NP, CW), jnp.float32),
    mesh=_mesh,
    scratch_types=[
        pltpu.VMEM((NCHUNK, CH), jnp.int32),
        pltpu.VMEM((NCHUNK, CH), jnp.int32),
        [pltpu.VMEM((CH, CW), jnp.float32)] * NBUF,
        [pltpu.SemaphoreType.DMA] * NBUF,
        pltpu.VMEM_SHARED((NP, CW), jnp.float32),
    ],
    compiler_params=pltpu.CompilerParams(needs_layout_passes=False),
)
def _spmm_call(row_hbm, col_hbm, *args):
    tabs = args[:NPASS]
    out_hbm = args[NPASS]
    idxc, idxr, bufs, sems, acc = args[NPASS + 1:]
    c = lax.axis_index("c")
    s = lax.axis_index("s")
    w = c * 16 + s
    zero = jnp.zeros((16,), jnp.float32)

    pltpu.sync_copy(col_hbm.at[w], idxc)
    pltpu.sync_copy(row_hbm.at[w], idxr)

    for p, tab in enumerate(tabs):
        @pl.loop(0, CH)
        def _zrow(r):
            for l in range(CW // 16):
                bufs[0][r, pl.ds(l * 16, 16)] = zero

        for k in range(RPT // CH):
            pltpu.sync_copy(bufs[0], acc.at[pl.ds(s * RPT + k * CH, CH)])
        plsc.subcore_barrier()

        for b in range(NBUF):
            pltpu.async_copy(tab.at[idxc.at[b]], bufs[b], sems[b])

        @pl.loop(0, NFULL // NBUF)
        def _edges(g):
            for b in range(NBUF):
                j = g * NBUF + b
                pltpu.make_async_copy(tab.at[idxc.at[j]], bufs[b],
                                      sems[b]).wait()
                pltpu.sync_copy(bufs[b], acc.at[idxr.at[j]], add=True)

                @pl.when(j + NBUF < NCHUNK)
                def _issue():
                    pltpu.async_copy(tab.at[idxc.at[j + NBUF]],
                                     bufs[b], sems[b])

        for j in range(NFULL, NCHUNK):
            b = j % NBUF
            pltpu.make_async_copy(tab.at[idxc.at[j]], bufs[b],
                                  sems[b]).wait()
            pltpu.sync_copy(bufs[b], acc.at[idxr.at[j]], add=True)

        plsc.subcore_barrier()
        pltpu.sync_copy(acc.at[pl.ds(s * RPT, RPT)],
                        out_hbm.at[c, p, pl.ds(s * RPT, RPT)])
        plsc.subcore_barrier()


def _final_body(*refs):
    acc_ref = refs[0]
    feat_refs = refs[1:1 + NPASS]
    d0_ref, d1_ref, smp_ref, o_ref = refs[1 + NPASS:]
    a = acc_ref[...]
    nh = NPASS // 2
    mean = jnp.concatenate(
        [a[0, q] + a[1, q] + feat_refs[q][...] for q in range(nh)], axis=1)
    var = jnp.concatenate(
        [a[0, nh + q] + a[1, nh + q] + feat_refs[nh + q][...]
         for q in range(nh)], axis=1)
    deg = d0_ref[...] + d1_ref[...] + 1.0
    dh = lax.rsqrt(deg)
    dinv1 = 1.0 / deg
    out = mean * dh + smp_ref[...] * jnp.sqrt(var * dinv1)
    m = jnp.max(out, axis=1, keepdims=True)
    sh = out - m
    lse = jnp.log(jnp.sum(jnp.exp(sh), axis=1, keepdims=True))
    o_ref[...] = sh - lse


def _final_call(accs, feats, d0, d1, sample):
    qspec = pl.BlockSpec((BLK, CW), lambda i: (i, 0))
    dspec = pl.BlockSpec((BLK, 1), lambda i: (i, 0))
    return pl.pallas_call(
        _final_body,
        grid=(GRID,),
        in_specs=[pl.BlockSpec((2, NPASS, BLK, CW), lambda i: (0, 0, i, 0))]
                 + [qspec] * NPASS
                 + [dspec, dspec, pl.BlockSpec((BLK, D), lambda i: (i, 0))],
        out_specs=pl.BlockSpec((BLK, D), lambda i: (i, 0)),
        out_shape=jax.ShapeDtypeStruct((N, D), jnp.float32),
    )(accs, *feats, d0, d1, sample)


def kernel(X, A, W, W0m, b0m, W0v, b0v, W1m, b1m, W1v, b1v):
    del W
    pad = N + jnp.arange(EP - E, dtype=jnp.int32) % (NP - N)
    rowp = jnp.concatenate([A[0], pad])
    colp = jnp.concatenate([A[1], pad])

    deg2 = _deg_call(rowp)
    d0 = deg2[0].reshape(NP, 1)
    d1 = deg2[1].reshape(NP, 1)

    feats = _dense_call(
        X, W0m, b0m.reshape(1, D), W0v, b0v.reshape(1, D),
        W1m, b1m.reshape(1, D), W1v, b1v.reshape(1, D), d0, d1)

    accs = _spmm_call(rowp.reshape(NW, NCHUNK, CH),
                      colp.reshape(NW, NCHUNK, CH), *feats)

    sample = jnp.asarray(_SAMPLE)
    return _final_call(accs, feats, d0, d1, sample)
